# baseline (speedup 1.0000x reference)
import functools

import jax
import jax.numpy as jnp
from jax import lax
from jax.experimental import pallas as pl
from jax.experimental.pallas import tpu as pltpu
from jax.experimental.pallas import tpu_sc as plsc

_N = 10000
_NPAD = 10240
_E = 320000
_D = 128
_NW = 32
_CB = 128
_CH = 80
_EPT = _CH * _CB
_STRIPE = _NPAD // 16
_BM = 1024


def _prep_edges(ei):
    ei = ei.astype(jnp.int32)
    pad = jnp.full((_NW * _EPT - _E,), _N, jnp.int32)
    src = jnp.concatenate([ei[0], pad]).reshape(_NW * _CH, _CB)
    dst = jnp.concatenate([ei[1], pad]).reshape(_NW * _CH, _CB)
    return src, dst


def _sc_degrees(dsts):
    mesh = plsc.VectorSubcoreMesh(core_axis_name="c", subcore_axis_name="s")

    @functools.partial(
        pl.kernel,
        out_type=jax.ShapeDtypeStruct((2, _NPAD, _D), jnp.float32),
        mesh=mesh,
        scratch_types=[
            pltpu.VMEM_SHARED((_NPAD, _D), jnp.float32),
            pltpu.VMEM((_CH, _CB), jnp.int32),
            pltpu.VMEM((_CB, _D), jnp.float32),
            pltpu.VMEM((16, _D), jnp.float32),
        ],
    )
    def k(dst_hbm, out_hbm, dacc, dst_v, srcb, zb):
        c = lax.axis_index("c")
        s = lax.axis_index("s")
        wid = s * 2 + c

        one16 = jnp.ones((16,), jnp.float32)
        zero16 = jnp.zeros((16,), jnp.float32)

        def fill_src(r):
            def fill(i, _):
                for g in range(_D // 16):
                    v = one16 if g // 2 == r else zero16
                    srcb[i, pl.ds(g * 16, 16)] = v
                return 0
            lax.fori_loop(0, _CB, fill, 0)

        def fill_z(i, _):
            for g in range(_D // 16):
                zb[i, pl.ds(g * 16, 16)] = zero16
            return 0
        lax.fori_loop(0, 16, fill_z, 0)

        base = s * _STRIPE
        def zcp(i, _):
            pltpu.sync_copy(zb, dacc.at[pl.ds(base + i * 16, 16)])
            return 0
        lax.fori_loop(0, _STRIPE // 16, zcp, 0)
        plsc.subcore_barrier()

        for r in range(3):
            fill_src(r)
            pltpu.sync_copy(
                dst_hbm.at[pl.ds((r * _NW + wid) * _CH, _CH)], dst_v)

            def body(j, _):
                pltpu.sync_copy(srcb, dacc.at[dst_v.at[j]], add=True)
                return 0
            lax.fori_loop(0, _CH, body, 0)
        plsc.subcore_barrier()

        pltpu.sync_copy(dacc.at[pl.ds(base, _STRIPE)],
                        out_hbm.at[c, pl.ds(base, _STRIPE)])

    return k(dsts)


_NCH0 = 112
_NCH1 = 48
_NG = 8


def _sc_gather_scatter_add(hp, srcp, dstp):
    mesh = plsc.VectorSubcoreMesh(core_axis_name="c", subcore_axis_name="s")

    @functools.partial(
        pl.kernel,
        out_type=jax.ShapeDtypeStruct((2, _NPAD, _D), jnp.float32),
        mesh=mesh,
        scratch_types=[
            pltpu.VMEM_SHARED((_NPAD, _D), jnp.float32),
            pltpu.VMEM((_NG, _CB), jnp.int32),
            pltpu.VMEM((_NG, _CB), jnp.int32),
            pltpu.VMEM((_NG, _CB), jnp.int32),
            pltpu.VMEM((_NG, _CB), jnp.int32),
            pltpu.VMEM((_CB, _D), jnp.float32),
            pltpu.VMEM((_CB, _D), jnp.float32),
            pltpu.VMEM((16, _D), jnp.float32),
            pltpu.SemaphoreType.DMA,
            pltpu.SemaphoreType.DMA,
            pltpu.SemaphoreType.DMA,
        ],
    )
    def k(hp_hbm, src_hbm, dst_hbm, out_hbm, acc,
          srcA, dstA, srcB, dstB, rowsA, rowsB, zb, semA, semB, semI):
        c = lax.axis_index("c")
        s = lax.axis_index("s")
        nch = jnp.where(c == 0, _NCH0, _NCH1)
        ng = nch // _NG
        base_chunk = jnp.where(c == 0, s * _NCH0, 16 * _NCH0 + s * _NCH1)
        src_base = base_chunk

        @pl.when(nch > 0)
        def _():
            pltpu.sync_copy(src_hbm.at[pl.ds(src_base, _NG)], srcA)
            pltpu.sync_copy(dst_hbm.at[pl.ds(base_chunk, _NG)], dstA)
            pltpu.async_copy(hp_hbm.at[srcA.at[0]], rowsA, semA)

        zero16 = jnp.zeros((16,), jnp.float32)

        def fill_z(i, _):
            for j in range(_D // 16):
                zb[i, pl.ds(j * 16, 16)] = zero16
            return 0
        lax.fori_loop(0, 16, fill_z, 0)

        base = s * _STRIPE
        def zcp(i, _):
            pltpu.sync_copy(zb, acc.at[pl.ds(base + i * 16, 16)])
            return 0
        lax.fori_loop(0, _STRIPE // 16, zcp, 0)
        plsc.subcore_barrier()

        rows_bufs = (rowsA, rowsB)
        sems = (semA, semB)

        def group_body(gi, srcCur, dstCur, srcNxt, dstNxt):
            @pl.when(gi + 1 < ng)
            def _():
                off = (gi + 1) * _NG
                pltpu.async_copy(
                    src_hbm.at[pl.ds(src_base + off, _NG)], srcNxt, semI)
                pltpu.async_copy(
                    dst_hbm.at[pl.ds(base_chunk + off, _NG)], dstNxt, semI)

            for j in range(_NG):
                rows_cur, sem_cur = rows_bufs[j % 2], sems[j % 2]
                rows_nxt, sem_nxt = rows_bufs[1 - j % 2], sems[1 - j % 2]
                pltpu.make_async_copy(
                    hp_hbm.at[srcCur.at[j]], rows_cur, sem_cur).wait()

                @pl.when(gi * _NG + j + 1 < nch)
                def _(j=j, srcCur=srcCur, srcNxt=srcNxt, dstNxt=dstNxt,
                      rows_nxt=rows_nxt, sem_nxt=sem_nxt):
                    if j == _NG - 1:
                        pltpu.make_async_copy(
                            src_hbm.at[pl.ds(0, _NG)], srcNxt, semI).wait()
                        pltpu.make_async_copy(
                            dst_hbm.at[pl.ds(0, _NG)], dstNxt, semI).wait()
                        idxr = srcNxt.at[0]
                    else:
                        idxr = srcCur.at[j + 1]
                    pltpu.async_copy(hp_hbm.at[idxr], rows_nxt, sem_nxt)

                pltpu.sync_copy(rows_cur, acc.at[dstCur.at[j]], add=True)

        def pair_body(gp, _):
            group_body(2 * gp, srcA, dstA, srcB, dstB)
            group_body(2 * gp + 1, srcB, dstB, srcA, dstA)
            return 0
        lax.fori_loop(0, ng // 2, pair_body, 0)
        plsc.subcore_barrier()

        pltpu.sync_copy(acc.at[pl.ds(base, _STRIPE)],
                        out_hbm.at[c, pl.ds(base, _STRIPE)])

    return k(hp, srcp, dstp)


def _dinv(degp_blk, r):
    deg = degp_blk[0, :, r * 32] + degp_blk[1, :, r * 32] + 1.0
    return lax.rsqrt(jnp.maximum(deg, 1e-12))


def _dot(a, b):
    return lax.dot(a, b, precision=lax.Precision.HIGHEST,
                   preferred_element_type=jnp.float32)


_DEG_SPEC = pl.BlockSpec((2, _BM, _D), lambda i: (0, i, 0))
_ROW_SPEC = pl.BlockSpec((_BM, _D), lambda i: (i, 0))
_W_SPEC = pl.BlockSpec((_D, _D), lambda i: (0, 0))
_B_SPEC = pl.BlockSpec((1, _D), lambda i: (0, 0))
_P_SPEC = pl.BlockSpec((2, _BM, _D), lambda i: (0, i, 0))
_GRID = (_NPAD // _BM,)


def _tc1_body(degp, x_ref, w0, w1, w2, o0, o1, o2):
    x = x_ref[...]
    dp = degp[...]
    for r, (w, o) in enumerate(((w0, o0), (w1, o1), (w2, o2))):
        dinv = _dinv(dp, r)[:, None]
        o[...] = _dot(x, w[...]) * dinv


def _tc1(degp, xp, ws):
    return pl.pallas_call(
        _tc1_body,
        grid=_GRID,
        in_specs=[_DEG_SPEC, _ROW_SPEC, _W_SPEC, _W_SPEC, _W_SPEC],
        out_specs=[_ROW_SPEC] * 3,
        out_shape=[jax.ShapeDtypeStruct((_NPAD, _D), jnp.float32)] * 3,
    )(degp, xp, *ws)


def _tc2_body(degp, p0, h0, b0, w0, p1, h1, b1, w1, p2, h2, b2, w2,
              o0, o1, o2):
    dp = degp[...]
    for r, (p, h, b, w, o) in enumerate((
            (p0, h0, b0, w0, o0), (p1, h1, b1, w1, o1),
            (p2, h2, b2, w2, o2))):
        dinv = _dinv(dp, r)[:, None]
        pv = p[...]
        x2 = jnp.maximum(dinv * (pv[0] + pv[1] + h[...]) + b[...], 0.0)
        o[...] = _dot(x2, w[...]) * dinv


def _tc2(degp, ps, hs, bs, ws):
    args = []
    for r in range(3):
        args += [ps[r], hs[r], bs[r], ws[r]]
    return pl.pallas_call(
        _tc2_body,
        grid=_GRID,
        in_specs=[_DEG_SPEC] + [_P_SPEC, _ROW_SPEC, _B_SPEC, _W_SPEC] * 3,
        out_specs=[_ROW_SPEC] * 3,
        out_shape=[jax.ShapeDtypeStruct((_NPAD, _D), jnp.float32)] * 3,
    )(degp, *args)


def _tc3_body(degp, aw, ab, p0, g0, b0, p1, g1, b1, p2, g2, b2, wout, aout):
    dp = degp[...]
    awv = aw[...]
    abv = ab[0, 0]
    outs, logits = [], []
    for r, (p, g, b) in enumerate(((p0, g0, b0), (p1, g1, b1), (p2, g2, b2))):
        dinv = _dinv(dp, r)[:, None]
        pv = p[...]
        o = dinv * (pv[0] + pv[1] + g[...]) + b[...]
        outs.append(o)
        logits.append(jnp.sum(o * awv, axis=1, keepdims=True) + abv)
    m = jnp.maximum(jnp.maximum(logits[0], logits[1]), logits[2])
    es = [jnp.exp(l - m) for l in logits]
    tot = es[0] + es[1] + es[2]
    wts = [e / tot for e in es]
    wout[...] = wts[0] * outs[0] + wts[1] * outs[1] + wts[2] * outs[2]
    aout[...] = jnp.concatenate(wts, axis=1)


def _tc3(degp, ps, gs, bs, attn_w, attn_b):
    args = []
    for r in range(3):
        args += [ps[r], gs[r], bs[r]]
    return pl.pallas_call(
        _tc3_body,
        grid=_GRID,
        in_specs=[_DEG_SPEC, _B_SPEC, pl.BlockSpec((1, 1), lambda i: (0, 0))]
                 + [_P_SPEC, _ROW_SPEC, _B_SPEC] * 3,
        out_specs=[_ROW_SPEC, pl.BlockSpec((_BM, 3), lambda i: (i, 0))],
        out_shape=[jax.ShapeDtypeStruct((_NPAD, _D), jnp.float32),
                   jax.ShapeDtypeStruct((_NPAD, 3), jnp.float32)],
    )(degp, attn_w, attn_b, *args)


def kernel(x, edge_index_ppi, edge_index_path, edge_index_go,
           W_ppi1, b_ppi1, W_ppi2, b_ppi2, W_path1, b_path1, W_path2, b_path2,
           W_go1, b_go1, W_go2, b_go2, attn_w, attn_b):
    xp = jnp.pad(x, ((0, _NPAD - _N), (0, 0)))
    edges = [_prep_edges(e)
             for e in (edge_index_ppi, edge_index_path, edge_index_go)]
    dsts = jnp.concatenate([d for (_s, d) in edges], axis=0)

    degp = _sc_degrees(dsts)

    h1s = _tc1(degp, xp, (W_ppi1, W_path1, W_go1))
    p1s = [_sc_gather_scatter_add(h1s[r], edges[r][0], edges[r][1])
           for r in range(3)]

    b1s = [b.reshape(1, _D) for b in (b_ppi1, b_path1, b_go1)]
    gs = _tc2(degp, p1s, h1s, b1s, (W_ppi2, W_path2, W_go2))
    p2s = [_sc_gather_scatter_add(gs[r], edges[r][0], edges[r][1])
           for r in range(3)]

    b2s = [b.reshape(1, _D) for b in (b_ppi2, b_path2, b_go2)]
    weighted, attn = _tc3(degp, p2s, gs, b2s,
                          attn_w.reshape(1, _D), attn_b.reshape(1, 1))
    return weighted[:_N], attn[:_N].reshape(_N, 3, 1)


# speedup vs baseline: 1.1634x; 1.1634x over previous
import functools

import jax
import jax.numpy as jnp
from jax import lax
from jax.experimental import pallas as pl
from jax.experimental.pallas import tpu as pltpu
from jax.experimental.pallas import tpu_sc as plsc

_N = 10000
_NPAD = 10240
_E = 320000
_D = 128
_NW = 32
_CB = 128
_CH = 80
_EPT = _CH * _CB
_STRIPE = _NPAD // 16
_BM = 1024


def _prep_edges(ei):
    ei = ei.astype(jnp.int32)
    pad = jnp.full((_NW * _EPT - _E,), _N, jnp.int32)
    src = jnp.concatenate([ei[0], pad]).reshape(_NW * _CH, _CB)
    dst = jnp.concatenate([ei[1], pad]).reshape(_NW * _CH, _CB)
    return src, dst


def _sc_degrees(dsts):
    mesh = plsc.VectorSubcoreMesh(core_axis_name="c", subcore_axis_name="s")

    @functools.partial(
        pl.kernel,
        out_type=jax.ShapeDtypeStruct((2, _NPAD, _D), jnp.float32),
        mesh=mesh,
        scratch_types=[
            pltpu.VMEM_SHARED((_NPAD, _D), jnp.float32),
            pltpu.VMEM((_CH, _CB), jnp.int32),
            pltpu.VMEM((_CB, _D), jnp.float32),
            pltpu.VMEM((16, _D), jnp.float32),
        ],
    )
    def k(dst_hbm, out_hbm, dacc, dst_v, srcb, zb):
        c = lax.axis_index("c")
        s = lax.axis_index("s")
        wid = s * 2 + c

        one16 = jnp.ones((16,), jnp.float32)
        zero16 = jnp.zeros((16,), jnp.float32)

        def fill_src(r):
            def fill(i, _):
                for g in range(_D // 16):
                    v = one16 if g // 2 == r else zero16
                    srcb[i, pl.ds(g * 16, 16)] = v
                return 0
            lax.fori_loop(0, _CB, fill, 0)

        def fill_z(i, _):
            for g in range(_D // 16):
                zb[i, pl.ds(g * 16, 16)] = zero16
            return 0
        lax.fori_loop(0, 16, fill_z, 0)

        base = s * _STRIPE
        def zcp(i, _):
            pltpu.sync_copy(zb, dacc.at[pl.ds(base + i * 16, 16)])
            return 0
        lax.fori_loop(0, _STRIPE // 16, zcp, 0)
        plsc.subcore_barrier()

        for r in range(3):
            fill_src(r)
            pltpu.sync_copy(
                dst_hbm.at[pl.ds((r * _NW + wid) * _CH, _CH)], dst_v)

            def body(j, _):
                pltpu.sync_copy(srcb, dacc.at[dst_v.at[j]], add=True)
                return 0
            lax.fori_loop(0, _CH, body, 0)
        plsc.subcore_barrier()

        pltpu.sync_copy(dacc.at[pl.ds(base, _STRIPE)],
                        out_hbm.at[c, pl.ds(base, _STRIPE)])

    return k(dsts)


_NCH0 = 144
_NCH1 = 16
_NG = 8


def _sc_gather_scatter_add(hp, srcp, dstp):
    mesh = plsc.VectorSubcoreMesh(core_axis_name="c", subcore_axis_name="s")

    @functools.partial(
        pl.kernel,
        out_type=jax.ShapeDtypeStruct((2, _NPAD, _D), jnp.float32),
        mesh=mesh,
        scratch_types=[
            pltpu.VMEM_SHARED((_NPAD, _D), jnp.float32),
            pltpu.VMEM((_NG, _CB), jnp.int32),
            pltpu.VMEM((_NG, _CB), jnp.int32),
            pltpu.VMEM((_NG, _CB), jnp.int32),
            pltpu.VMEM((_NG, _CB), jnp.int32),
            pltpu.VMEM((_CB, _D), jnp.float32),
            pltpu.VMEM((_CB, _D), jnp.float32),
            pltpu.VMEM((16, _D), jnp.float32),
            pltpu.SemaphoreType.DMA,
            pltpu.SemaphoreType.DMA,
            pltpu.SemaphoreType.DMA,
        ],
    )
    def k(hp_hbm, src_hbm, dst_hbm, out_hbm, acc,
          srcA, dstA, srcB, dstB, rowsA, rowsB, zb, semA, semB, semI):
        c = lax.axis_index("c")
        s = lax.axis_index("s")
        nch = jnp.where(c == 0, _NCH0, _NCH1)
        ng = nch // _NG
        base_chunk = jnp.where(c == 0, s * _NCH0, 16 * _NCH0 + s * _NCH1)
        src_base = base_chunk

        @pl.when(nch > 0)
        def _():
            pltpu.sync_copy(src_hbm.at[pl.ds(src_base, _NG)], srcA)
            pltpu.sync_copy(dst_hbm.at[pl.ds(base_chunk, _NG)], dstA)
            pltpu.async_copy(hp_hbm.at[srcA.at[0]], rowsA, semA)

        zero16 = jnp.zeros((16,), jnp.float32)

        def fill_z(i, _):
            for j in range(_D // 16):
                zb[i, pl.ds(j * 16, 16)] = zero16
            return 0
        lax.fori_loop(0, 16, fill_z, 0)

        base = s * _STRIPE
        def zcp(i, _):
            pltpu.sync_copy(zb, acc.at[pl.ds(base + i * 16, 16)])
            return 0
        lax.fori_loop(0, _STRIPE // 16, zcp, 0)
        plsc.subcore_barrier()

        rows_bufs = (rowsA, rowsB)
        sems = (semA, semB)

        def group_body(gi, srcCur, dstCur, srcNxt, dstNxt):
            @pl.when(gi + 1 < ng)
            def _():
                off = (gi + 1) * _NG
                pltpu.async_copy(
                    src_hbm.at[pl.ds(src_base + off, _NG)], srcNxt, semI)
                pltpu.async_copy(
                    dst_hbm.at[pl.ds(base_chunk + off, _NG)], dstNxt, semI)

            for j in range(_NG):
                rows_cur, sem_cur = rows_bufs[j % 2], sems[j % 2]
                rows_nxt, sem_nxt = rows_bufs[1 - j % 2], sems[1 - j % 2]
                pltpu.make_async_copy(
                    hp_hbm.at[srcCur.at[j]], rows_cur, sem_cur).wait()

                @pl.when(gi * _NG + j + 1 < nch)
                def _(j=j, srcCur=srcCur, srcNxt=srcNxt, dstNxt=dstNxt,
                      rows_nxt=rows_nxt, sem_nxt=sem_nxt):
                    if j == _NG - 1:
                        pltpu.make_async_copy(
                            src_hbm.at[pl.ds(0, _NG)], srcNxt, semI).wait()
                        pltpu.make_async_copy(
                            dst_hbm.at[pl.ds(0, _NG)], dstNxt, semI).wait()
                        idxr = srcNxt.at[0]
                    else:
                        idxr = srcCur.at[j + 1]
                    pltpu.async_copy(hp_hbm.at[idxr], rows_nxt, sem_nxt)

                pltpu.sync_copy(rows_cur, acc.at[dstCur.at[j]], add=True)

        def pair_body(gp, _):
            group_body(2 * gp, srcA, dstA, srcB, dstB)
            group_body(2 * gp + 1, srcB, dstB, srcA, dstA)
            return 0
        lax.fori_loop(0, ng // 2, pair_body, 0)
        plsc.subcore_barrier()

        pltpu.sync_copy(acc.at[pl.ds(base, _STRIPE)],
                        out_hbm.at[c, pl.ds(base, _STRIPE)])

    return k(hp, srcp, dstp)


def _dinv(degp_blk, r):
    deg = degp_blk[0, :, r * 32] + degp_blk[1, :, r * 32] + 1.0
    return lax.rsqrt(jnp.maximum(deg, 1e-12))


def _dot(a, b):
    return lax.dot(a, b, precision=lax.Precision.HIGHEST,
                   preferred_element_type=jnp.float32)


_DEG_SPEC = pl.BlockSpec((2, _BM, _D), lambda i: (0, i, 0))
_ROW_SPEC = pl.BlockSpec((_BM, _D), lambda i: (i, 0))
_W_SPEC = pl.BlockSpec((_D, _D), lambda i: (0, 0))
_B_SPEC = pl.BlockSpec((1, _D), lambda i: (0, 0))
_P_SPEC = pl.BlockSpec((2, _BM, _D), lambda i: (0, i, 0))
_GRID = (_NPAD // _BM,)


def _tc1_body(degp, x_ref, w0, w1, w2, o0, o1, o2):
    x = x_ref[...]
    dp = degp[...]
    for r, (w, o) in enumerate(((w0, o0), (w1, o1), (w2, o2))):
        dinv = _dinv(dp, r)[:, None]
        o[...] = _dot(x, w[...]) * dinv


def _tc1(degp, xp, ws):
    return pl.pallas_call(
        _tc1_body,
        grid=_GRID,
        in_specs=[_DEG_SPEC, _ROW_SPEC, _W_SPEC, _W_SPEC, _W_SPEC],
        out_specs=[_ROW_SPEC] * 3,
        out_shape=[jax.ShapeDtypeStruct((_NPAD, _D), jnp.float32)] * 3,
    )(degp, xp, *ws)


def _tc2_body(degp, p0, h0, b0, w0, p1, h1, b1, w1, p2, h2, b2, w2,
              o0, o1, o2):
    dp = degp[...]
    for r, (p, h, b, w, o) in enumerate((
            (p0, h0, b0, w0, o0), (p1, h1, b1, w1, o1),
            (p2, h2, b2, w2, o2))):
        dinv = _dinv(dp, r)[:, None]
        pv = p[...]
        x2 = jnp.maximum(dinv * (pv[0] + pv[1] + h[...]) + b[...], 0.0)
        o[...] = _dot(x2, w[...]) * dinv


def _tc2(degp, ps, hs, bs, ws):
    args = []
    for r in range(3):
        args += [ps[r], hs[r], bs[r], ws[r]]
    return pl.pallas_call(
        _tc2_body,
        grid=_GRID,
        in_specs=[_DEG_SPEC] + [_P_SPEC, _ROW_SPEC, _B_SPEC, _W_SPEC] * 3,
        out_specs=[_ROW_SPEC] * 3,
        out_shape=[jax.ShapeDtypeStruct((_NPAD, _D), jnp.float32)] * 3,
    )(degp, *args)


def _tc3_body(degp, aw, ab, p0, g0, b0, p1, g1, b1, p2, g2, b2, wout, aout):
    dp = degp[...]
    awv = aw[...]
    abv = ab[0, 0]
    outs, logits = [], []
    for r, (p, g, b) in enumerate(((p0, g0, b0), (p1, g1, b1), (p2, g2, b2))):
        dinv = _dinv(dp, r)[:, None]
        pv = p[...]
        o = dinv * (pv[0] + pv[1] + g[...]) + b[...]
        outs.append(o)
        logits.append(jnp.sum(o * awv, axis=1, keepdims=True) + abv)
    m = jnp.maximum(jnp.maximum(logits[0], logits[1]), logits[2])
    es = [jnp.exp(l - m) for l in logits]
    tot = es[0] + es[1] + es[2]
    wts = [e / tot for e in es]
    wout[...] = wts[0] * outs[0] + wts[1] * outs[1] + wts[2] * outs[2]
    aout[...] = jnp.concatenate(wts, axis=1)


def _tc3(degp, ps, gs, bs, attn_w, attn_b):
    args = []
    for r in range(3):
        args += [ps[r], gs[r], bs[r]]
    return pl.pallas_call(
        _tc3_body,
        grid=_GRID,
        in_specs=[_DEG_SPEC, _B_SPEC, pl.BlockSpec((1, 1), lambda i: (0, 0))]
                 + [_P_SPEC, _ROW_SPEC, _B_SPEC] * 3,
        out_specs=[_ROW_SPEC, pl.BlockSpec((_BM, 3), lambda i: (i, 0))],
        out_shape=[jax.ShapeDtypeStruct((_NPAD, _D), jnp.float32),
                   jax.ShapeDtypeStruct((_NPAD, 3), jnp.float32)],
    )(degp, attn_w, attn_b, *args)


def kernel(x, edge_index_ppi, edge_index_path, edge_index_go,
           W_ppi1, b_ppi1, W_ppi2, b_ppi2, W_path1, b_path1, W_path2, b_path2,
           W_go1, b_go1, W_go2, b_go2, attn_w, attn_b):
    xp = jnp.pad(x, ((0, _NPAD - _N), (0, 0)))
    edges = [_prep_edges(e)
             for e in (edge_index_ppi, edge_index_path, edge_index_go)]
    dsts = jnp.concatenate([d for (_s, d) in edges], axis=0)

    degp = _sc_degrees(dsts)

    h1s = _tc1(degp, xp, (W_ppi1, W_path1, W_go1))
    p1s = [_sc_gather_scatter_add(h1s[r], edges[r][0], edges[r][1])
           for r in range(3)]

    b1s = [b.reshape(1, _D) for b in (b_ppi1, b_path1, b_go1)]
    gs = _tc2(degp, p1s, h1s, b1s, (W_ppi2, W_path2, W_go2))
    p2s = [_sc_gather_scatter_add(gs[r], edges[r][0], edges[r][1])
           for r in range(3)]

    b2s = [b.reshape(1, _D) for b in (b_ppi2, b_path2, b_go2)]
    weighted, attn = _tc3(degp, p2s, gs, b2s,
                          attn_w.reshape(1, _D), attn_b.reshape(1, 1))
    return weighted[:_N], attn[:_N].reshape(_N, 3, 1)
